# initial kernel scaffold (unmeasured)
import jax
import jax.numpy as jnp
from jax import lax
from jax.experimental import pallas as pl
from jax.experimental.pallas import tpu as pltpu

N_DEV = 8


def kernel(x, w_mat):
    m_per, k = x.shape
    _, n = w_mat.shape
    n_per = n // N_DEV
    m = m_per * N_DEV

    def body(
        x_ref,
        w_ref,
        out_ref,
        y_ref,
        comm_ref,
        my_amax_ref,
        amax_rx_ref,
        send_sems,
        recv_sems,
        amax_send_sems,
        amax_recv_sems,
    ):
        my_i = lax.axis_index("i")

        y = jnp.dot(x_ref[:, :], w_ref[:, :], preferred_element_type=jnp.float32)
        y_ref[:, :] = y.astype(jnp.bfloat16)
        local_amax = jnp.max(jnp.abs(y))
        my_amax_ref[0, :] = jnp.full((128,), local_amax, jnp.float32)

        rdmas = []
        for d in range(1, N_DEV):
            dst = (my_i + d) % N_DEV
            data = pltpu.make_async_remote_copy(
                src_ref=y_ref.at[:, pl.ds(dst * n_per, n_per)],
                dst_ref=comm_ref.at[d - 1],
                send_sem=send_sems.at[d - 1],
                recv_sem=recv_sems.at[d - 1],
                device_id=(dst,),
                device_id_type=pl.DeviceIdType.MESH,
            )
            data.start()
            am = pltpu.make_async_remote_copy(
                src_ref=my_amax_ref,
                dst_ref=amax_rx_ref.at[pl.ds(d - 1, 1)],
                send_sem=amax_send_sems.at[d - 1],
                recv_sem=amax_recv_sems.at[d - 1],
                device_id=(dst,),
                device_id_type=pl.DeviceIdType.MESH,
            )
            am.start()
            rdmas.append((data, am))

        for data, am in rdmas:
            data.wait()
            am.wait()

        gmax = jnp.maximum(local_amax, jnp.max(amax_rx_ref[:, :]))
        scale = gmax / 127.0
        inv_scale = 127.0 / gmax

        def qdq(block):
            q = jnp.clip(
                jnp.round(block.astype(jnp.float32) * inv_scale), -127.0, 127.0
            )
            return q * scale

        out_ref[pl.ds(my_i * m_per, m_per), :] = qdq(
            y_ref[:, pl.ds(my_i * n_per, n_per)]
        )
        for d in range(1, N_DEV):
            origin = (my_i - d) % N_DEV
            out_ref[pl.ds(origin * m_per, m_per), :] = qdq(comm_ref[d - 1])

    return pl.pallas_call(
        body,
        out_shape=jax.ShapeDtypeStruct((m, n_per), jnp.float32),
        in_specs=[
            pl.BlockSpec(memory_space=pltpu.VMEM),
            pl.BlockSpec(memory_space=pltpu.VMEM),
        ],
        out_specs=pl.BlockSpec(memory_space=pltpu.VMEM),
        scratch_shapes=[
            pltpu.VMEM((m_per, n), jnp.bfloat16),
            pltpu.VMEM((N_DEV - 1, m_per, n_per), jnp.bfloat16),
            pltpu.VMEM((1, 128), jnp.float32),
            pltpu.VMEM((N_DEV - 1, 128), jnp.float32),
            pltpu.SemaphoreType.DMA((N_DEV - 1,)),
            pltpu.SemaphoreType.DMA((N_DEV - 1,)),
            pltpu.SemaphoreType.DMA((N_DEV - 1,)),
            pltpu.SemaphoreType.DMA((N_DEV - 1,)),
        ],
        compiler_params=pltpu.CompilerParams(collective_id=0),
    )(x, w_mat)


# baseline (device time: 53172 ns/iter reference)
import jax
import jax.numpy as jnp
from jax import lax
from jax.experimental import pallas as pl
from jax.experimental.pallas import tpu as pltpu

N_DEV = 8


def kernel(x, w_mat):
    m_per, k = x.shape
    _, n = w_mat.shape
    n_per = n // N_DEV
    m = m_per * N_DEV

    def body(
        x_ref,
        w_ref,
        out_ref,
        y_ref,
        comm_ref,
        my_amax_ref,
        amax_rx_ref,
        send_sems,
        recv_sems,
        amax_send_sems,
        amax_recv_sems,
    ):
        my_i = lax.axis_index("i")

        xb = x_ref[:, :].astype(jnp.bfloat16)
        wb = w_ref[:, :].astype(jnp.bfloat16)
        y = jnp.dot(xb, wb, preferred_element_type=jnp.float32)
        y_ref[:, :] = y.astype(jnp.bfloat16)
        local_amax = jnp.max(jnp.abs(y))
        my_amax_ref[0, :] = jnp.full((128,), local_amax, jnp.float32)

        rdmas = []
        for d in range(1, N_DEV):
            dst = (my_i + d) % N_DEV
            data = pltpu.make_async_remote_copy(
                src_ref=y_ref.at[:, pl.ds(dst * n_per, n_per)],
                dst_ref=comm_ref.at[d - 1],
                send_sem=send_sems.at[d - 1],
                recv_sem=recv_sems.at[d - 1],
                device_id=(dst,),
                device_id_type=pl.DeviceIdType.MESH,
            )
            data.start()
            am = pltpu.make_async_remote_copy(
                src_ref=my_amax_ref,
                dst_ref=amax_rx_ref.at[pl.ds(d - 1, 1)],
                send_sem=amax_send_sems.at[d - 1],
                recv_sem=amax_recv_sems.at[d - 1],
                device_id=(dst,),
                device_id_type=pl.DeviceIdType.MESH,
            )
            am.start()
            rdmas.append((data, am))

        for data, am in rdmas:
            data.wait()
            am.wait()

        gmax = jnp.maximum(local_amax, jnp.max(amax_rx_ref[:, :]))
        scale = gmax / 127.0
        inv_scale = 127.0 / gmax

        def qdq(block):
            q = jnp.clip(
                jnp.round(block.astype(jnp.float32) * inv_scale), -127.0, 127.0
            )
            return q * scale

        out_ref[pl.ds(my_i * m_per, m_per), :] = qdq(
            y_ref[:, pl.ds(my_i * n_per, n_per)]
        )
        for d in range(1, N_DEV):
            origin = (my_i - d) % N_DEV
            out_ref[pl.ds(origin * m_per, m_per), :] = qdq(comm_ref[d - 1])

    return pl.pallas_call(
        body,
        out_shape=jax.ShapeDtypeStruct((m, n_per), jnp.float32),
        in_specs=[
            pl.BlockSpec(memory_space=pltpu.VMEM),
            pl.BlockSpec(memory_space=pltpu.VMEM),
        ],
        out_specs=pl.BlockSpec(memory_space=pltpu.VMEM),
        scratch_shapes=[
            pltpu.VMEM((m_per, n), jnp.bfloat16),
            pltpu.VMEM((N_DEV - 1, m_per, n_per), jnp.bfloat16),
            pltpu.VMEM((1, 128), jnp.float32),
            pltpu.VMEM((N_DEV - 1, 128), jnp.float32),
            pltpu.SemaphoreType.DMA((N_DEV - 1,)),
            pltpu.SemaphoreType.DMA((N_DEV - 1,)),
            pltpu.SemaphoreType.DMA((N_DEV - 1,)),
            pltpu.SemaphoreType.DMA((N_DEV - 1,)),
        ],
        compiler_params=pltpu.CompilerParams(
            vmem_limit_bytes=110 * 1024 * 1024,
        ),
    )(x, w_mat)


# device time: 50368 ns/iter; 1.0557x vs baseline; 1.0557x over previous
import functools

import jax
import jax.numpy as jnp
from jax import lax
from jax.experimental import pallas as pl
from jax.experimental.pallas import tpu as pltpu

N_DEV = 8


def kernel(x, w_mat):
    m_per, k = x.shape
    _, n = w_mat.shape
    n_per = n // N_DEV
    m = m_per * N_DEV

    def body(
        x_ref,
        w_ref,
        out_ref,
        y_ref,
        comm_ref,
        my_amax_ref,
        amax_rx_ref,
        send_sems,
        recv_sems,
        amax_send_sems,
        amax_recv_sems,
    ):
        my_i = lax.axis_index("i")

        xb = x_ref[:, :].astype(jnp.bfloat16)

        rdmas = []
        amaxes = []
        own_y = None
        for d in range(1, N_DEV + 1):
            dst = (my_i + d) % N_DEV
            wb = w_ref[:, pl.ds(dst * n_per, n_per)].astype(jnp.bfloat16)
            yj = jnp.dot(xb, wb, preferred_element_type=jnp.float32)
            amaxes.append(jnp.max(jnp.abs(yj)))
            if d < N_DEV:
                y_ref[:, pl.ds(dst * n_per, n_per)] = yj.astype(jnp.bfloat16)
                data = pltpu.make_async_remote_copy(
                    src_ref=y_ref.at[:, pl.ds(dst * n_per, n_per)],
                    dst_ref=comm_ref.at[d - 1],
                    send_sem=send_sems.at[d - 1],
                    recv_sem=recv_sems.at[d - 1],
                    device_id=(dst,),
                    device_id_type=pl.DeviceIdType.MESH,
                )
                data.start()
                rdmas.append(data)
            else:
                own_y = yj

        local_amax = functools.reduce(jnp.maximum, amaxes)
        my_amax_ref[0, :] = jnp.full((128,), local_amax, jnp.float32)
        am_rdmas = []
        for d in range(1, N_DEV):
            dst = (my_i + d) % N_DEV
            am = pltpu.make_async_remote_copy(
                src_ref=my_amax_ref,
                dst_ref=amax_rx_ref.at[pl.ds(d - 1, 1)],
                send_sem=amax_send_sems.at[d - 1],
                recv_sem=amax_recv_sems.at[d - 1],
                device_id=(dst,),
                device_id_type=pl.DeviceIdType.MESH,
            )
            am.start()
            am_rdmas.append(am)

        for am in am_rdmas:
            am.wait()

        gmax = jnp.maximum(local_amax, jnp.max(amax_rx_ref[:, :]))
        scale = gmax / 127.0
        inv_scale = 127.0 / gmax

        def qdq(block):
            q = jnp.clip(
                jnp.round(block.astype(jnp.float32) * inv_scale), -127.0, 127.0
            )
            return q * scale

        out_ref[pl.ds(my_i * m_per, m_per), :] = qdq(own_y)
        for d in range(1, N_DEV):
            rdmas[d - 1].wait()
            origin = (my_i - d) % N_DEV
            out_ref[pl.ds(origin * m_per, m_per), :] = qdq(comm_ref[d - 1])

    return pl.pallas_call(
        body,
        out_shape=jax.ShapeDtypeStruct((m, n_per), jnp.float32),
        in_specs=[
            pl.BlockSpec(memory_space=pltpu.VMEM),
            pl.BlockSpec(memory_space=pltpu.VMEM),
        ],
        out_specs=pl.BlockSpec(memory_space=pltpu.VMEM),
        scratch_shapes=[
            pltpu.VMEM((m_per, n), jnp.bfloat16),
            pltpu.VMEM((N_DEV - 1, m_per, n_per), jnp.bfloat16),
            pltpu.VMEM((1, 128), jnp.float32),
            pltpu.VMEM((N_DEV - 1, 128), jnp.float32),
            pltpu.SemaphoreType.DMA((N_DEV - 1,)),
            pltpu.SemaphoreType.DMA((N_DEV - 1,)),
            pltpu.SemaphoreType.DMA((N_DEV - 1,)),
            pltpu.SemaphoreType.DMA((N_DEV - 1,)),
        ],
        compiler_params=pltpu.CompilerParams(
            vmem_limit_bytes=110 * 1024 * 1024,
        ),
    )(x, w_mat)


# device time: 45758 ns/iter; 1.1620x vs baseline; 1.1007x over previous
import functools

import jax
import jax.numpy as jnp
from jax import lax
from jax.experimental import pallas as pl
from jax.experimental.pallas import tpu as pltpu

N_DEV = 8


def kernel(x, w_mat):
    m_per, k = x.shape
    _, n = w_mat.shape
    n_per = n // N_DEV
    m = m_per * N_DEV

    def body(
        x_ref,
        w_ref,
        out_ref,
        y_ref,
        comm_ref,
        my_amax_ref,
        amax_rx_ref,
        send_sems,
        recv_sems,
        amax_send_sems,
        amax_recv_sems,
    ):
        my_i = lax.axis_index("i")

        barrier_sem = pltpu.get_barrier_semaphore()
        for d in range(1, N_DEV):
            pl.semaphore_signal(
                barrier_sem,
                inc=1,
                device_id=((my_i + d) % N_DEV,),
                device_id_type=pl.DeviceIdType.MESH,
            )

        xb = x_ref[:, :].astype(jnp.bfloat16)

        rdmas = []
        amaxes = []
        own_y = None
        for d in range(1, N_DEV + 1):
            dst = (my_i + d) % N_DEV
            wb = w_ref[:, pl.ds(dst * n_per, n_per)].astype(jnp.bfloat16)
            yj = jnp.dot(xb, wb, preferred_element_type=jnp.float32)
            amaxes.append(jnp.max(jnp.abs(yj)))
            if d < N_DEV:
                y_ref[:, pl.ds(dst * n_per, n_per)] = yj.astype(jnp.bfloat16)
                if d == 1:
                    pl.semaphore_wait(barrier_sem, N_DEV - 1)
                data = pltpu.make_async_remote_copy(
                    src_ref=y_ref.at[:, pl.ds(dst * n_per, n_per)],
                    dst_ref=comm_ref.at[d - 1],
                    send_sem=send_sems.at[d - 1],
                    recv_sem=recv_sems.at[d - 1],
                    device_id=(dst,),
                    device_id_type=pl.DeviceIdType.MESH,
                )
                data.start()
                rdmas.append(data)
            else:
                own_y = yj

        local_amax = functools.reduce(jnp.maximum, amaxes)
        my_amax_ref[0, :] = jnp.full((128,), local_amax, jnp.float32)
        am_rdmas = []
        for d in range(1, N_DEV):
            dst = (my_i + d) % N_DEV
            am = pltpu.make_async_remote_copy(
                src_ref=my_amax_ref,
                dst_ref=amax_rx_ref.at[pl.ds(d - 1, 1)],
                send_sem=amax_send_sems.at[d - 1],
                recv_sem=amax_recv_sems.at[d - 1],
                device_id=(dst,),
                device_id_type=pl.DeviceIdType.MESH,
            )
            am.start()
            am_rdmas.append(am)

        for am in am_rdmas:
            am.wait()

        gmax = jnp.maximum(local_amax, jnp.max(amax_rx_ref[:, :]))
        scale = gmax / 127.0
        inv_scale = 127.0 / gmax

        def qdq(block):
            q = jnp.clip(
                jnp.round(block.astype(jnp.float32) * inv_scale), -127.0, 127.0
            )
            return q * scale

        out_ref[pl.ds(my_i * m_per, m_per), :] = qdq(own_y)
        for d in range(1, N_DEV):
            rdmas[d - 1].wait()
            origin = (my_i - d) % N_DEV
            out_ref[pl.ds(origin * m_per, m_per), :] = qdq(comm_ref[d - 1])

    return pl.pallas_call(
        body,
        out_shape=jax.ShapeDtypeStruct((m, n_per), jnp.float32),
        in_specs=[
            pl.BlockSpec(memory_space=pltpu.VMEM),
            pl.BlockSpec(memory_space=pltpu.VMEM),
        ],
        out_specs=pl.BlockSpec(memory_space=pltpu.VMEM),
        scratch_shapes=[
            pltpu.VMEM((m_per, n), jnp.bfloat16),
            pltpu.VMEM((N_DEV - 1, m_per, n_per), jnp.bfloat16),
            pltpu.VMEM((1, 128), jnp.float32),
            pltpu.VMEM((N_DEV - 1, 128), jnp.float32),
            pltpu.SemaphoreType.DMA((N_DEV - 1,)),
            pltpu.SemaphoreType.DMA((N_DEV - 1,)),
            pltpu.SemaphoreType.DMA((N_DEV - 1,)),
            pltpu.SemaphoreType.DMA((N_DEV - 1,)),
        ],
        compiler_params=pltpu.CompilerParams(
            vmem_limit_bytes=110 * 1024 * 1024,
            collective_id=0,
        ),
    )(x, w_mat)


# device time: 43908 ns/iter; 1.2110x vs baseline; 1.0421x over previous
import functools

import jax
import jax.numpy as jnp
from jax import lax
from jax.experimental import pallas as pl
from jax.experimental.pallas import tpu as pltpu

N_DEV = 8


def kernel(x, w_mat):
    m_per, k = x.shape
    _, n = w_mat.shape
    n_per = n // N_DEV
    m = m_per * N_DEV

    def body(
        x_ref,
        w_ref,
        out_ref,
        y_ref,
        comm_ref,
        my_amax_ref,
        amax_rx_ref,
        send_sems,
        recv_sems,
        amax_send_sems,
        amax_recv_sems,
    ):
        my_i = lax.axis_index("i")

        barrier_sem = pltpu.get_barrier_semaphore()
        for d in range(1, N_DEV):
            pl.semaphore_signal(
                barrier_sem,
                inc=1,
                device_id=((my_i + d) % N_DEV,),
                device_id_type=pl.DeviceIdType.MESH,
            )

        xb = x_ref[:, :].astype(jnp.bfloat16)

        rdmas = []
        amaxes = []
        own_y = None
        for d in range(1, N_DEV + 1):
            dst = (my_i + d) % N_DEV
            wb = w_ref[:, pl.ds(dst * n_per, n_per)].astype(jnp.bfloat16)
            yj = jnp.dot(xb, wb, preferred_element_type=jnp.float32)
            amaxes.append(jnp.max(jnp.abs(yj)))
            if d < N_DEV:
                y_ref[:, pl.ds(dst * n_per, n_per)] = yj.astype(jnp.bfloat16)
                if d == 1:
                    pl.semaphore_wait(barrier_sem, N_DEV - 1)
                data = pltpu.make_async_remote_copy(
                    src_ref=y_ref.at[:, pl.ds(dst * n_per, n_per)],
                    dst_ref=comm_ref.at[d - 1],
                    send_sem=send_sems.at[d - 1],
                    recv_sem=recv_sems.at[d - 1],
                    device_id=(dst,),
                    device_id_type=pl.DeviceIdType.MESH,
                )
                data.start()
                rdmas.append(data)
            else:
                own_y = yj

        local_amax = functools.reduce(jnp.maximum, amaxes)
        my_amax_ref[0, :] = jnp.full((128,), local_amax, jnp.float32)
        am_rdmas = []
        for d in range(1, N_DEV):
            dst = (my_i + d) % N_DEV
            am = pltpu.make_async_remote_copy(
                src_ref=my_amax_ref,
                dst_ref=amax_rx_ref.at[pl.ds(d - 1, 1)],
                send_sem=amax_send_sems.at[d - 1],
                recv_sem=amax_recv_sems.at[d - 1],
                device_id=(dst,),
                device_id_type=pl.DeviceIdType.MESH,
            )
            am.start()
            am_rdmas.append(am)

        for am in am_rdmas:
            am.wait()

        gmax = jnp.maximum(local_amax, jnp.max(amax_rx_ref[:, :]))
        scale = gmax / 127.0
        inv_scale = 127.0 / gmax

        def qdq(block):
            q = jnp.clip(
                jnp.round(block.astype(jnp.float32) * inv_scale), -127.0, 127.0
            )
            return (q * scale).astype(jnp.bfloat16)

        out_ref[pl.ds(my_i * m_per, m_per), :] = qdq(own_y)
        for d in range(1, N_DEV):
            rdmas[d - 1].wait()
            origin = (my_i - d) % N_DEV
            out_ref[pl.ds(origin * m_per, m_per), :] = qdq(comm_ref[d - 1])

    return pl.pallas_call(
        body,
        out_shape=jax.ShapeDtypeStruct((m, n_per), jnp.bfloat16),
        in_specs=[
            pl.BlockSpec(memory_space=pltpu.VMEM),
            pl.BlockSpec(memory_space=pltpu.VMEM),
        ],
        out_specs=pl.BlockSpec(memory_space=pltpu.VMEM),
        scratch_shapes=[
            pltpu.VMEM((m_per, n), jnp.bfloat16),
            pltpu.VMEM((N_DEV - 1, m_per, n_per), jnp.bfloat16),
            pltpu.VMEM((1, 128), jnp.float32),
            pltpu.VMEM((N_DEV - 1, 128), jnp.float32),
            pltpu.SemaphoreType.DMA((N_DEV - 1,)),
            pltpu.SemaphoreType.DMA((N_DEV - 1,)),
            pltpu.SemaphoreType.DMA((N_DEV - 1,)),
            pltpu.SemaphoreType.DMA((N_DEV - 1,)),
        ],
        compiler_params=pltpu.CompilerParams(
            vmem_limit_bytes=110 * 1024 * 1024,
            collective_id=0,
        ),
    )(x, w_mat)
